# baseline (speedup 1.0000x reference)
import functools

import jax
import jax.numpy as jnp
from jax import lax
from jax.experimental import pallas as pl
from jax.experimental.pallas import tpu as pltpu
from jax.experimental.pallas import tpu_sc as plsc

VOCAB = 1000000
HIDDEN = 64
NUM_IDS = 4096 * 200

NC = 2
NS = 16
NW = NC * NS
B_PER_W = NUM_IDS // NW

G = 128
K = 4
CHUNK = G * K
N_CHUNKS = B_PER_W // CHUNK
NBUF = 2

TB = 4096
T_GRID = (VOCAB + TB - 1) // TB


def _detile_block(src_ref, out_ref):
    eye = jnp.eye(HIDDEN, dtype=jnp.float32)
    t = lax.dot_general(
        src_ref[...], eye,
        dimension_numbers=(((0,), (0,)), ((), ())),
        preferred_element_type=jnp.float32,
        precision=lax.Precision.HIGHEST,
    )
    out_ref[...] = jnp.concatenate([t, t], axis=1)


_detile = pl.pallas_call(
    _detile_block,
    grid=(T_GRID,),
    in_specs=[pl.BlockSpec((HIDDEN, TB), lambda g: (0, g))],
    out_specs=pl.BlockSpec((TB, 128), lambda g: (g, 0)),
    out_shape=jax.ShapeDtypeStruct((VOCAB, 128), jnp.float32),
)


def _make_gather():
    mesh = plsc.VectorSubcoreMesh(core_axis_name="c", subcore_axis_name="s")

    @functools.partial(
        pl.kernel,
        mesh=mesh,
        out_type=jax.ShapeDtypeStruct((NUM_IDS, 128), jnp.float32),
        scratch_types=[
            pltpu.VMEM((B_PER_W,), jnp.int32),
            pltpu.VMEM((NBUF * CHUNK, HIDDEN), jnp.float32),
            pltpu.SemaphoreType.DMA,
        ],
        compiler_params=pltpu.CompilerParams(use_tc_tiling_on_sc=False),
    )
    def gather_kernel(idx_hbm, table_hbm, out_hbm, idx_v, rows_v, gsem):
        wid = lax.axis_index("s") * NC + lax.axis_index("c")
        base = wid * B_PER_W

        pltpu.sync_copy(idx_hbm.at[pl.ds(base, B_PER_W)], idx_v)

        def fire(i, slot):
            voff = slot * CHUNK
            for j in range(K):
                pltpu.async_copy(
                    table_hbm.at[idx_v.at[pl.ds(i * CHUNK + j * G, G)]],
                    rows_v.at[pl.ds(voff + j * G, G)],
                    gsem,
                )

        def drain_and_store(i, slot):
            off = base + i * CHUNK
            voff = slot * CHUNK
            for j in range(K):
                pltpu.make_async_copy(
                    table_hbm.at[idx_v.at[pl.ds(i * CHUNK + j * G, G)]],
                    rows_v.at[pl.ds(voff + j * G, G)],
                    gsem,
                ).wait()
            pltpu.sync_copy(rows_v.at[pl.ds(voff, CHUNK)],
                            out_hbm.at[pl.ds(off, CHUNK), pl.ds(0, HIDDEN)])

        fire(0, 0)

        def body(i, _):
            @pl.when(i + 1 < N_CHUNKS)
            def _():
                fire(i + 1, lax.rem(i + 1, NBUF))

            drain_and_store(i, lax.rem(i, NBUF))
            return 0

        lax.fori_loop(0, N_CHUNKS, body, 0)

    return gather_kernel


_gather = _make_gather()


def kernel(input_ids, emb):
    idx = input_ids.reshape(-1).astype(jnp.int32) * 2
    table_lin = _detile(emb.T).reshape(2 * VOCAB, HIDDEN)
    h = _gather(idx, table_lin)
    h = h[:, :HIDDEN].reshape(input_ids.shape + (HIDDEN,))
    return (h, h)


# speedup vs baseline: 1.1196x; 1.1196x over previous
import functools

import jax
import jax.numpy as jnp
from jax import lax
from jax.experimental import pallas as pl
from jax.experimental.pallas import tpu as pltpu
from jax.experimental.pallas import tpu_sc as plsc

VOCAB = 1000000
HIDDEN = 64
NUM_IDS = 4096 * 200

NC = 2
NS = 16
NW = NC * NS
B_PER_W = NUM_IDS // NW

G = 128
K = 4
CHUNK = G * K
N_CHUNKS = B_PER_W // CHUNK
NBUF = 2

TB = 4096
T_GRID = (VOCAB + TB - 1) // TB


def _detile_block(src_ref, out_ref):
    t = src_ref[...].T
    out_ref[...] = jnp.concatenate([t, t], axis=1)


_detile = pl.pallas_call(
    _detile_block,
    grid=(T_GRID,),
    in_specs=[pl.BlockSpec((HIDDEN, TB), lambda g: (0, g))],
    out_specs=pl.BlockSpec((TB, 128), lambda g: (g, 0)),
    out_shape=jax.ShapeDtypeStruct((VOCAB, 128), jnp.float32),
)


def _make_gather():
    mesh = plsc.VectorSubcoreMesh(core_axis_name="c", subcore_axis_name="s")

    @functools.partial(
        pl.kernel,
        mesh=mesh,
        out_type=jax.ShapeDtypeStruct((NUM_IDS, 128), jnp.float32),
        scratch_types=[
            pltpu.VMEM((B_PER_W,), jnp.int32),
            pltpu.VMEM((NBUF * CHUNK, HIDDEN), jnp.float32),
            pltpu.SemaphoreType.DMA,
        ],
        compiler_params=pltpu.CompilerParams(use_tc_tiling_on_sc=False),
    )
    def gather_kernel(idx_hbm, table_hbm, out_hbm, idx_v, rows_v, gsem):
        wid = lax.axis_index("s") * NC + lax.axis_index("c")
        base = wid * B_PER_W

        pltpu.sync_copy(idx_hbm.at[pl.ds(base, B_PER_W)], idx_v)

        def fire(i, slot):
            voff = slot * CHUNK
            for j in range(K):
                pltpu.async_copy(
                    table_hbm.at[idx_v.at[pl.ds(i * CHUNK + j * G, G)]],
                    rows_v.at[pl.ds(voff + j * G, G)],
                    gsem,
                )

        def drain_and_store(i, slot):
            off = base + i * CHUNK
            voff = slot * CHUNK
            for j in range(K):
                pltpu.make_async_copy(
                    table_hbm.at[idx_v.at[pl.ds(i * CHUNK + j * G, G)]],
                    rows_v.at[pl.ds(voff + j * G, G)],
                    gsem,
                ).wait()
            pltpu.sync_copy(rows_v.at[pl.ds(voff, CHUNK)],
                            out_hbm.at[pl.ds(off, CHUNK), pl.ds(0, HIDDEN)])

        fire(0, 0)

        def body(i, _):
            @pl.when(i + 1 < N_CHUNKS)
            def _():
                fire(i + 1, lax.rem(i + 1, NBUF))

            drain_and_store(i, lax.rem(i, NBUF))
            return 0

        lax.fori_loop(0, N_CHUNKS, body, 0)

    return gather_kernel


_gather = _make_gather()


def kernel(input_ids, emb):
    idx = input_ids.reshape(-1).astype(jnp.int32) * 2
    table_lin = _detile(emb.T).reshape(2 * VOCAB, HIDDEN)
    h = _gather(idx, table_lin)
    h = h[:, :HIDDEN].reshape(input_ids.shape + (HIDDEN,))
    return (h, h)


# speedup vs baseline: 1.2172x; 1.0871x over previous
import functools

import jax
import jax.numpy as jnp
from jax import lax
from jax.experimental import pallas as pl
from jax.experimental.pallas import tpu as pltpu
from jax.experimental.pallas import tpu_sc as plsc

VOCAB = 1000000
HIDDEN = 64
NUM_IDS = 4096 * 200

NC = 2
NS = 16
NW = NC * NS
B_PER_W = NUM_IDS // NW

G = 128
K = 4
CHUNK = G * K
N_CHUNKS = B_PER_W // CHUNK
NBUF = 2

TB = 8192
T_GRID = (VOCAB + TB - 1) // TB


def _detile_block(src_ref, out_ref):
    t = src_ref[...].T
    out_ref[...] = jnp.concatenate([t, t], axis=1)


_detile = pl.pallas_call(
    _detile_block,
    grid=(T_GRID,),
    in_specs=[pl.BlockSpec((HIDDEN, TB), lambda g: (0, g))],
    out_specs=pl.BlockSpec((TB, 128), lambda g: (g, 0)),
    out_shape=jax.ShapeDtypeStruct((VOCAB, 128), jnp.float32),
)


def _make_gather():
    mesh = plsc.VectorSubcoreMesh(core_axis_name="c", subcore_axis_name="s")

    @functools.partial(
        pl.kernel,
        mesh=mesh,
        out_type=jax.ShapeDtypeStruct((NUM_IDS, 128), jnp.float32),
        scratch_types=[
            pltpu.VMEM((B_PER_W,), jnp.int32),
            pltpu.VMEM((NBUF * CHUNK, HIDDEN), jnp.float32),
            pltpu.SemaphoreType.DMA,
        ],
        compiler_params=pltpu.CompilerParams(use_tc_tiling_on_sc=False),
    )
    def gather_kernel(idx_hbm, table_hbm, out_hbm, idx_v, rows_v, gsem):
        wid = lax.axis_index("s") * NC + lax.axis_index("c")
        base = wid * B_PER_W

        pltpu.sync_copy(idx_hbm.at[pl.ds(base, B_PER_W)], idx_v)

        def fire(i, slot):
            voff = slot * CHUNK
            for j in range(K):
                pltpu.async_copy(
                    table_hbm.at[idx_v.at[pl.ds(i * CHUNK + j * G, G)]],
                    rows_v.at[pl.ds(voff + j * G, G)],
                    gsem,
                )

        def drain_and_store(i, slot):
            off = base + i * CHUNK
            voff = slot * CHUNK
            for j in range(K):
                pltpu.make_async_copy(
                    table_hbm.at[idx_v.at[pl.ds(i * CHUNK + j * G, G)]],
                    rows_v.at[pl.ds(voff + j * G, G)],
                    gsem,
                ).wait()
            pltpu.sync_copy(rows_v.at[pl.ds(voff, CHUNK)],
                            out_hbm.at[pl.ds(off, CHUNK), pl.ds(0, HIDDEN)])

        fire(0, 0)

        def body(i, _):
            @pl.when(i + 1 < N_CHUNKS)
            def _():
                fire(i + 1, lax.rem(i + 1, NBUF))

            drain_and_store(i, lax.rem(i, NBUF))
            return 0

        lax.fori_loop(0, N_CHUNKS, body, 0)

    return gather_kernel


_gather = _make_gather()


def kernel(input_ids, emb):
    idx = input_ids.reshape(-1).astype(jnp.int32) * 2
    table_lin = _detile(emb.T).reshape(2 * VOCAB, HIDDEN)
    h = _gather(idx, table_lin)
    h = h[:, :HIDDEN].reshape(input_ids.shape + (HIDDEN,))
    return (h, h)


# speedup vs baseline: 1.2709x; 1.0441x over previous
import functools

import jax
import jax.numpy as jnp
from jax import lax
from jax.experimental import pallas as pl
from jax.experimental.pallas import tpu as pltpu
from jax.experimental.pallas import tpu_sc as plsc

VOCAB = 1000000
HIDDEN = 64
NUM_IDS = 4096 * 200

NC = 2
NS = 16
NW = NC * NS
B_PER_W = NUM_IDS // NW

G = 128
K = 4
CHUNK = G * K
N_CHUNKS = B_PER_W // CHUNK
NBUF = 2

TB = 16384
T_GRID = (VOCAB + TB - 1) // TB


def _detile_block(src_ref, out_ref):
    t = src_ref[...].T
    out_ref[...] = jnp.concatenate([t, t], axis=1)


_detile = pl.pallas_call(
    _detile_block,
    grid=(T_GRID,),
    in_specs=[pl.BlockSpec((HIDDEN, TB), lambda g: (0, g))],
    out_specs=pl.BlockSpec((TB, 128), lambda g: (g, 0)),
    out_shape=jax.ShapeDtypeStruct((VOCAB, 128), jnp.float32),
)


def _make_gather():
    mesh = plsc.VectorSubcoreMesh(core_axis_name="c", subcore_axis_name="s")

    @functools.partial(
        pl.kernel,
        mesh=mesh,
        out_type=jax.ShapeDtypeStruct((NUM_IDS, 128), jnp.float32),
        scratch_types=[
            pltpu.VMEM((B_PER_W,), jnp.int32),
            pltpu.VMEM((NBUF * CHUNK, HIDDEN), jnp.float32),
            pltpu.SemaphoreType.DMA,
        ],
        compiler_params=pltpu.CompilerParams(use_tc_tiling_on_sc=False),
    )
    def gather_kernel(idx_hbm, table_hbm, out_hbm, idx_v, rows_v, gsem):
        wid = lax.axis_index("s") * NC + lax.axis_index("c")
        base = wid * B_PER_W

        pltpu.sync_copy(idx_hbm.at[pl.ds(base, B_PER_W)], idx_v)

        def fire(i, slot):
            voff = slot * CHUNK
            for j in range(K):
                pltpu.async_copy(
                    table_hbm.at[idx_v.at[pl.ds(i * CHUNK + j * G, G)]],
                    rows_v.at[pl.ds(voff + j * G, G)],
                    gsem,
                )

        def drain_and_store(i, slot):
            off = base + i * CHUNK
            voff = slot * CHUNK
            for j in range(K):
                pltpu.make_async_copy(
                    table_hbm.at[idx_v.at[pl.ds(i * CHUNK + j * G, G)]],
                    rows_v.at[pl.ds(voff + j * G, G)],
                    gsem,
                ).wait()
            pltpu.sync_copy(rows_v.at[pl.ds(voff, CHUNK)],
                            out_hbm.at[pl.ds(off, CHUNK), pl.ds(0, HIDDEN)])

        fire(0, 0)

        def body(i, _):
            @pl.when(i + 1 < N_CHUNKS)
            def _():
                fire(i + 1, lax.rem(i + 1, NBUF))

            drain_and_store(i, lax.rem(i, NBUF))
            return 0

        lax.fori_loop(0, N_CHUNKS, body, 0)

    return gather_kernel


_gather = _make_gather()


def kernel(input_ids, emb):
    idx = input_ids.reshape(-1).astype(jnp.int32) * 2
    table_lin = _detile(emb.T).reshape(2 * VOCAB, HIDDEN)
    h = _gather(idx, table_lin)
    h = h[:, :HIDDEN].reshape(input_ids.shape + (HIDDEN,))
    return (h, h)


# speedup vs baseline: 1.2889x; 1.0142x over previous
import functools

import jax
import jax.numpy as jnp
from jax import lax
from jax.experimental import pallas as pl
from jax.experimental.pallas import tpu as pltpu
from jax.experimental.pallas import tpu_sc as plsc

VOCAB = 1000000
HIDDEN = 64
NUM_IDS = 4096 * 200

NC = 2
NS = 16
NW = NC * NS
B_PER_W = NUM_IDS // NW

G = 128
K = 4
CHUNK = G * K
N_CHUNKS = B_PER_W // CHUNK
NBUF = 2

TB = 32768
T_GRID = (VOCAB + TB - 1) // TB


def _detile_block(src_ref, out_ref):
    t = src_ref[...].T
    out_ref[...] = jnp.concatenate([t, t], axis=1)


_detile = pl.pallas_call(
    _detile_block,
    grid=(T_GRID,),
    in_specs=[pl.BlockSpec((HIDDEN, TB), lambda g: (0, g))],
    out_specs=pl.BlockSpec((TB, 128), lambda g: (g, 0)),
    out_shape=jax.ShapeDtypeStruct((VOCAB, 128), jnp.float32),
    compiler_params=pltpu.CompilerParams(vmem_limit_bytes=100 * 1024 * 1024),
)


def _make_gather():
    mesh = plsc.VectorSubcoreMesh(core_axis_name="c", subcore_axis_name="s")

    @functools.partial(
        pl.kernel,
        mesh=mesh,
        out_type=jax.ShapeDtypeStruct((NUM_IDS, 128), jnp.float32),
        scratch_types=[
            pltpu.VMEM((B_PER_W,), jnp.int32),
            pltpu.VMEM((NBUF * CHUNK, HIDDEN), jnp.float32),
            pltpu.SemaphoreType.DMA,
        ],
        compiler_params=pltpu.CompilerParams(use_tc_tiling_on_sc=False),
    )
    def gather_kernel(idx_hbm, table_hbm, out_hbm, idx_v, rows_v, gsem):
        wid = lax.axis_index("s") * NC + lax.axis_index("c")
        base = wid * B_PER_W

        pltpu.sync_copy(idx_hbm.at[pl.ds(base, B_PER_W)], idx_v)

        def fire(i, slot):
            voff = slot * CHUNK
            for j in range(K):
                pltpu.async_copy(
                    table_hbm.at[idx_v.at[pl.ds(i * CHUNK + j * G, G)]],
                    rows_v.at[pl.ds(voff + j * G, G)],
                    gsem,
                )

        def drain_and_store(i, slot):
            off = base + i * CHUNK
            voff = slot * CHUNK
            for j in range(K):
                pltpu.make_async_copy(
                    table_hbm.at[idx_v.at[pl.ds(i * CHUNK + j * G, G)]],
                    rows_v.at[pl.ds(voff + j * G, G)],
                    gsem,
                ).wait()
            pltpu.sync_copy(rows_v.at[pl.ds(voff, CHUNK)],
                            out_hbm.at[pl.ds(off, CHUNK), pl.ds(0, HIDDEN)])

        fire(0, 0)

        def body(i, _):
            @pl.when(i + 1 < N_CHUNKS)
            def _():
                fire(i + 1, lax.rem(i + 1, NBUF))

            drain_and_store(i, lax.rem(i, NBUF))
            return 0

        lax.fori_loop(0, N_CHUNKS, body, 0)

    return gather_kernel


_gather = _make_gather()


def kernel(input_ids, emb):
    idx = input_ids.reshape(-1).astype(jnp.int32) * 2
    table_lin = _detile(emb.T).reshape(2 * VOCAB, HIDDEN)
    h = _gather(idx, table_lin)
    h = h[:, :HIDDEN].reshape(input_ids.shape + (HIDDEN,))
    return (h, h)
